# baseline (speedup 1.0000x reference)
import dataclasses

import jax
import jax.numpy as jnp
from jax import lax
from jax.experimental import pallas as pl
from jax.experimental.pallas import tpu as pltpu
from jax.experimental.pallas import tpu_sc as plsc

N_NODES = 10000
N_EDGES = 160000
DIM = 256
LANES = 16
N_WORKERS = 32
B_W = N_EDGES // N_WORKERS
CHUNK = 96
NF = B_W // CHUNK
TAIL = B_W - NF * CHUNK
TAIL_GROUPS = -(-TAIL // LANES)


_GDN = lax.GatherDimensionNumbers(offset_dims=(), collapsed_slice_dims=(0,),
                                  start_index_map=(0,))


def _lane_perm(v, idx):
    return lax.gather(v, idx[:, None], _GDN, (1,),
                      mode=lax.GatherScatterMode.PROMISE_IN_BOUNDS)


def _lane_bcast_sum(v, perms):
    for p in perms:
        v = v + _lane_perm(v, p)
    return v


def _dot_group(rows_s, rows_d, i0, lane, perms):
    outv = jnp.zeros((LANES,), jnp.float32)
    for e in range(LANES):
        i = i0 + e
        acc = rows_s[i, pl.ds(0, LANES)] * rows_d[i, pl.ds(0, LANES)]
        for cd in range(1, DIM // LANES):
            acc = acc + (rows_s[i, pl.ds(cd * LANES, LANES)]
                         * rows_d[i, pl.ds(cd * LANES, LANES)])
        tot = _lane_bcast_sum(acc, perms)
        outv = jnp.where(lane == e, tot, outv)
    return 1.0 / (1.0 + jnp.exp(-outv))


def _sc_body(z_hbm, srci_hbm, dsti_hbm, out_hbm,
             idx_s, idx_d, rows_s0, rows_d0, rows_s1, rows_d1, out0, out1,
             sem_gs0, sem_gd0, sem_gs1, sem_gd1, sem_o0, sem_o1):
    rows_s = (rows_s0, rows_s1)
    rows_d = (rows_d0, rows_d1)
    out_v = (out0, out1)
    sem_gs = (sem_gs0, sem_gs1)
    sem_gd = (sem_gd0, sem_gd1)
    sem_o = (sem_o0, sem_o1)

    cid = lax.axis_index("c")
    sid = lax.axis_index("s")
    wid = sid * 2 + cid
    base_e = wid * B_W
    lane = lax.iota(jnp.int32, LANES)
    perms = tuple(lane ^ h for h in (1, 2, 4, 8))

    pltpu.sync_copy(srci_hbm.at[pl.ds(base_e, B_W)], idx_s)
    pltpu.sync_copy(dsti_hbm.at[pl.ds(base_e, B_W)], idx_d)

    def start_gather(k, b):
        off = pl.multiple_of(k * CHUNK, 8)
        pltpu.async_copy(z_hbm.at[idx_s.at[pl.ds(off, CHUNK)]],
                         rows_s[b], sem_gs[b])
        pltpu.async_copy(z_hbm.at[idx_d.at[pl.ds(off, CHUNK)]],
                         rows_d[b], sem_gd[b])

    def wait_gather(b):
        pltpu.make_async_copy(z_hbm.at[pl.ds(0, CHUNK)], rows_s[b],
                              sem_gs[b]).wait()
        pltpu.make_async_copy(z_hbm.at[pl.ds(0, CHUNK)], rows_d[b],
                              sem_gd[b]).wait()

    def wait_store(b):
        pltpu.make_async_copy(out_v[b], out_hbm.at[pl.ds(0, CHUNK)],
                              sem_o[b]).wait()

    start_gather(0, 0)
    start_gather(1, 1)

    @pl.loop(0, NF, step=2)
    def _pair(k):
        for b in range(2):
            kk = k + b
            wait_gather(b)
            @pl.when(kk >= 2)
            def _():
                wait_store(b)

            @pl.loop(0, CHUNK, step=LANES)
            def _group(i0):
                out_v[b][pl.ds(i0, LANES)] = _dot_group(
                    rows_s[b], rows_d[b], i0, lane, perms)

            off = pl.multiple_of(base_e + kk * CHUNK, 8)
            pltpu.async_copy(out_v[b], out_hbm.at[pl.ds(off, CHUNK)],
                             sem_o[b])

            @pl.when(kk + 2 < NF)
            def _():
                start_gather(kk + 2, b)

    wait_store(0)
    wait_store(1)

    t_off = NF * CHUNK
    g_s = pltpu.async_copy(z_hbm.at[idx_s.at[pl.ds(t_off, TAIL)]],
                           rows_s[0].at[pl.ds(0, TAIL)], sem_gs[0])
    g_d = pltpu.async_copy(z_hbm.at[idx_d.at[pl.ds(t_off, TAIL)]],
                           rows_d[0].at[pl.ds(0, TAIL)], sem_gd[0])
    g_s.wait()
    g_d.wait()
    for g in range(TAIL_GROUPS):
        out_v[0][pl.ds(g * LANES, LANES)] = _dot_group(
            rows_s[0], rows_d[0], g * LANES, lane, perms)
    pltpu.sync_copy(out_v[0].at[pl.ds(0, TAIL)],
                    out_hbm.at[pl.ds(base_e + t_off, TAIL)])


def _make_sc_kernel():
    mesh = plsc.VectorSubcoreMesh(core_axis_name="c", subcore_axis_name="s")
    cp = pltpu.CompilerParams()
    if "needs_layout_passes" in pltpu.CompilerParams.__dataclass_fields__:
        cp = dataclasses.replace(cp, needs_layout_passes=False)
    return pl.kernel(
        _sc_body,
        out_type=jax.ShapeDtypeStruct((N_EDGES,), jnp.float32),
        mesh=mesh,
        scratch_types=[
            pltpu.VMEM((B_W,), jnp.int32),
            pltpu.VMEM((B_W,), jnp.int32),
            pltpu.VMEM((CHUNK, DIM), jnp.float32),
            pltpu.VMEM((CHUNK, DIM), jnp.float32),
            pltpu.VMEM((CHUNK, DIM), jnp.float32),
            pltpu.VMEM((CHUNK, DIM), jnp.float32),
            pltpu.VMEM((CHUNK,), jnp.float32),
            pltpu.VMEM((CHUNK,), jnp.float32),
            pltpu.SemaphoreType.DMA,
            pltpu.SemaphoreType.DMA,
            pltpu.SemaphoreType.DMA,
            pltpu.SemaphoreType.DMA,
            pltpu.SemaphoreType.DMA,
            pltpu.SemaphoreType.DMA,
        ],
        compiler_params=cp,
    )


_sc_kernel = _make_sc_kernel()


def kernel(z, edge_index):
    ei = edge_index.astype(jnp.int32)
    return _sc_kernel(z, ei[0], ei[1])


# speedup vs baseline: 1.3549x; 1.3549x over previous
import dataclasses

import jax
import jax.numpy as jnp
from jax import lax
from jax.experimental import pallas as pl
from jax.experimental.pallas import tpu as pltpu
from jax.experimental.pallas import tpu_sc as plsc

N_NODES = 10000
N_EDGES = 160000
DIM = 256
LANES = 16
N_WORKERS = 32
B_W = N_EDGES // N_WORKERS
CHUNK = 64
NBUF = 3
NF = B_W // CHUNK
TAIL = B_W - NF * CHUNK
TAIL_GROUPS = -(-TAIL // LANES)


def _dot_group(rows_s, rows_d, out_ref, i0):
    out_ref[pl.ds(i0, LANES)] = jnp.zeros((LANES,), jnp.float32)
    base_idx = jnp.full((LANES,), i0, jnp.int32)
    for e in range(LANES):
        i = i0 + e
        acc = rows_s[i, pl.ds(0, LANES)] * rows_d[i, pl.ds(0, LANES)]
        for cd in range(1, DIM // LANES):
            acc = acc + (rows_s[i, pl.ds(cd * LANES, LANES)]
                         * rows_d[i, pl.ds(cd * LANES, LANES)])
        plsc.addupdate_scatter(out_ref, [base_idx + e], acc)
    v = out_ref[pl.ds(i0, LANES)]
    out_ref[pl.ds(i0, LANES)] = 1.0 / (1.0 + jnp.exp(-v))


def _sc_body(z_hbm, srci_hbm, dsti_hbm, out_hbm,
             idx_s, idx_d,
             rows_s0, rows_d0, rows_s1, rows_d1, rows_s2, rows_d2,
             out0, out1, out2,
             sem_gs0, sem_gd0, sem_gs1, sem_gd1, sem_gs2, sem_gd2,
             sem_o0, sem_o1, sem_o2):
    rows_s = (rows_s0, rows_s1, rows_s2)
    rows_d = (rows_d0, rows_d1, rows_d2)
    out_v = (out0, out1, out2)
    sem_gs = (sem_gs0, sem_gs1, sem_gs2)
    sem_gd = (sem_gd0, sem_gd1, sem_gd2)
    sem_o = (sem_o0, sem_o1, sem_o2)

    cid = lax.axis_index("c")
    sid = lax.axis_index("s")
    wid = sid * 2 + cid
    base_e = wid * B_W

    pltpu.sync_copy(srci_hbm.at[pl.ds(base_e, B_W)], idx_s)
    pltpu.sync_copy(dsti_hbm.at[pl.ds(base_e, B_W)], idx_d)

    def start_gather(k, b):
        off = pl.multiple_of(k * CHUNK, 8)
        pltpu.async_copy(z_hbm.at[idx_s.at[pl.ds(off, CHUNK)]],
                         rows_s[b], sem_gs[b])
        pltpu.async_copy(z_hbm.at[idx_d.at[pl.ds(off, CHUNK)]],
                         rows_d[b], sem_gd[b])

    def wait_gather(b):
        pltpu.make_async_copy(z_hbm.at[pl.ds(0, CHUNK)], rows_s[b],
                              sem_gs[b]).wait()
        pltpu.make_async_copy(z_hbm.at[pl.ds(0, CHUNK)], rows_d[b],
                              sem_gd[b]).wait()

    def wait_store(b):
        pltpu.make_async_copy(out_v[b], out_hbm.at[pl.ds(0, CHUNK)],
                              sem_o[b]).wait()

    for b in range(NBUF):
        start_gather(b, b)

    @pl.loop(0, NF, step=NBUF)
    def _ring(k):
        for b in range(NBUF):
            kk = k + b
            wait_gather(b)
            @pl.when(kk >= NBUF)
            def _():
                wait_store(b)

            @pl.loop(0, CHUNK, step=LANES)
            def _group(i0):
                _dot_group(rows_s[b], rows_d[b], out_v[b], i0)

            off = pl.multiple_of(base_e + kk * CHUNK, 8)
            pltpu.async_copy(out_v[b], out_hbm.at[pl.ds(off, CHUNK)],
                             sem_o[b])

            @pl.when(kk + NBUF < NF)
            def _():
                start_gather(kk + NBUF, b)

    for b in range(NBUF):
        wait_store(b)

    t_off = NF * CHUNK
    g_s = pltpu.async_copy(z_hbm.at[idx_s.at[pl.ds(t_off, TAIL)]],
                           rows_s[0].at[pl.ds(0, TAIL)], sem_gs[0])
    g_d = pltpu.async_copy(z_hbm.at[idx_d.at[pl.ds(t_off, TAIL)]],
                           rows_d[0].at[pl.ds(0, TAIL)], sem_gd[0])
    g_s.wait()
    g_d.wait()

    @pl.loop(0, TAIL_GROUPS * LANES, step=LANES)
    def _tail_group(i0):
        _dot_group(rows_s[0], rows_d[0], out_v[0], i0)

    pltpu.sync_copy(out_v[0].at[pl.ds(0, TAIL)],
                    out_hbm.at[pl.ds(base_e + t_off, TAIL)])


def _make_sc_kernel():
    mesh = plsc.VectorSubcoreMesh(core_axis_name="c", subcore_axis_name="s")
    cp = pltpu.CompilerParams()
    if "needs_layout_passes" in pltpu.CompilerParams.__dataclass_fields__:
        cp = dataclasses.replace(cp, needs_layout_passes=False)
    scratch_types = [
        pltpu.VMEM((B_W,), jnp.int32),
        pltpu.VMEM((B_W,), jnp.int32),
    ]
    for _ in range(NBUF):
        scratch_types.append(pltpu.VMEM((CHUNK, DIM), jnp.float32))
        scratch_types.append(pltpu.VMEM((CHUNK, DIM), jnp.float32))
    scratch_types += [pltpu.VMEM((CHUNK,), jnp.float32)] * NBUF
    scratch_types += [pltpu.SemaphoreType.DMA] * (3 * NBUF)
    return pl.kernel(
        _sc_body,
        out_type=jax.ShapeDtypeStruct((N_EDGES,), jnp.float32),
        mesh=mesh,
        scratch_types=scratch_types,
        compiler_params=cp,
    )


_sc_kernel = _make_sc_kernel()


def kernel(z, edge_index):
    ei = edge_index.astype(jnp.int32)
    return _sc_kernel(z, ei[0], ei[1])


# speedup vs baseline: 1.3989x; 1.0325x over previous
import dataclasses

import jax
import jax.numpy as jnp
from jax import lax
from jax.experimental import pallas as pl
from jax.experimental.pallas import tpu as pltpu
from jax.experimental.pallas import tpu_sc as plsc

N_NODES = 10000
N_EDGES = 160000
DIM = 256
DIMW = DIM // 2
LANES = 16
N_WORKERS = 32
B_W = N_EDGES // N_WORKERS
CHUNK = 32
NBUF = 3
NF = B_W // CHUNK
TAIL = B_W - NF * CHUNK
TAIL_GROUPS = -(-TAIL // LANES)
CONV_CHUNK = 40
N_CONV = N_NODES // CONV_CHUNK


def _dot_group(rows_s, rows_d, out_ref, i0):
    out_ref[pl.ds(i0, LANES)] = jnp.zeros((LANES,), jnp.float32)
    base_idx = jnp.full((LANES,), i0, jnp.int32)
    for e in range(LANES):
        i = i0 + e
        acc0 = acc1 = None
        for cd in range(DIMW // LANES):
            s = plsc.bitcast(rows_s[i, pl.ds(cd * LANES, LANES)],
                             jnp.bfloat16)
            d = plsc.bitcast(rows_d[i, pl.ds(cd * LANES, LANES)],
                             jnp.bfloat16)
            pa, pb = plsc.unpack(s * d, format=plsc.PackFormat.INTERLEAVED,
                                 preferred_element_type=jnp.float32)
            acc0 = pa if acc0 is None else acc0 + pa
            acc1 = pb if acc1 is None else acc1 + pb
        plsc.addupdate_scatter(out_ref, [base_idx + e], acc0 + acc1)
    v = out_ref[pl.ds(i0, LANES)]
    out_ref[pl.ds(i0, LANES)] = 1.0 / (1.0 + jnp.exp(-v))


def _sc_body(z_hbm, srci_hbm, dsti_hbm, out_hbm,
             zbf_sh, idx_s, idx_d, stage_f, stage_o,
             rows_s0, rows_d0, rows_s1, rows_d1, rows_s2, rows_d2,
             out0, out1, out2,
             sem_gs0, sem_gd0, sem_gs1, sem_gd1, sem_gs2, sem_gd2,
             sem_o0, sem_o1, sem_o2):
    rows_s = (rows_s0, rows_s1, rows_s2)
    rows_d = (rows_d0, rows_d1, rows_d2)
    out_v = (out0, out1, out2)
    sem_gs = (sem_gs0, sem_gs1, sem_gs2)
    sem_gd = (sem_gd0, sem_gd1, sem_gd2)
    sem_o = (sem_o0, sem_o1, sem_o2)

    cid = lax.axis_index("c")
    sid = lax.axis_index("s")
    wid = sid * 2 + cid
    base_e = wid * B_W

    n_conv = jnp.where(sid < N_CONV - (N_CONV // 16) * 16,
                       N_CONV // 16 + 1, N_CONV // 16)

    @pl.loop(0, n_conv)
    def _conv(j):
        r0 = pl.multiple_of((sid + j * 16) * CONV_CHUNK, 16)
        pltpu.sync_copy(z_hbm.at[pl.ds(r0, CONV_CHUNK)], stage_f)

        @pl.loop(0, CONV_CHUNK)
        def _conv_row(r):
            for cd in range(DIM // (2 * LANES)):
                a = stage_f[r, pl.ds(cd * 2 * LANES, LANES)]
                b = stage_f[r, pl.ds(cd * 2 * LANES + LANES, LANES)]
                p = plsc.pack(a, b, format=plsc.PackFormat.INTERLEAVED)
                stage_o[r, pl.ds(cd * LANES, LANES)] = plsc.bitcast(
                    p, jnp.int32)

        pltpu.sync_copy(stage_o, zbf_sh.at[pl.ds(r0, CONV_CHUNK)])
    plsc.subcore_barrier()

    pltpu.sync_copy(srci_hbm.at[pl.ds(base_e, B_W)], idx_s)
    pltpu.sync_copy(dsti_hbm.at[pl.ds(base_e, B_W)], idx_d)

    def start_gather(k, b):
        off = pl.multiple_of(k * CHUNK, 8)
        pltpu.async_copy(zbf_sh.at[idx_s.at[pl.ds(off, CHUNK)]],
                         rows_s[b], sem_gs[b])
        pltpu.async_copy(zbf_sh.at[idx_d.at[pl.ds(off, CHUNK)]],
                         rows_d[b], sem_gd[b])

    def wait_gather(b):
        pltpu.make_async_copy(zbf_sh.at[pl.ds(0, CHUNK)], rows_s[b],
                              sem_gs[b]).wait()
        pltpu.make_async_copy(zbf_sh.at[pl.ds(0, CHUNK)], rows_d[b],
                              sem_gd[b]).wait()

    def wait_store(b):
        pltpu.make_async_copy(out_v[b], out_hbm.at[pl.ds(0, CHUNK)],
                              sem_o[b]).wait()

    for b in range(NBUF):
        start_gather(b, b)

    @pl.loop(0, NF, step=NBUF)
    def _ring(k):
        for b in range(NBUF):
            kk = k + b
            wait_gather(b)
            @pl.when(kk >= NBUF)
            def _():
                wait_store(b)

            @pl.loop(0, CHUNK, step=LANES)
            def _group(i0):
                _dot_group(rows_s[b], rows_d[b], out_v[b], i0)

            off = pl.multiple_of(base_e + kk * CHUNK, 8)
            pltpu.async_copy(out_v[b], out_hbm.at[pl.ds(off, CHUNK)],
                             sem_o[b])

            @pl.when(kk + NBUF < NF)
            def _():
                start_gather(kk + NBUF, b)

    for b in range(NBUF):
        wait_store(b)

    t_off = NF * CHUNK
    g_s = pltpu.async_copy(zbf_sh.at[idx_s.at[pl.ds(t_off, TAIL)]],
                           rows_s[0].at[pl.ds(0, TAIL)], sem_gs[0])
    g_d = pltpu.async_copy(zbf_sh.at[idx_d.at[pl.ds(t_off, TAIL)]],
                           rows_d[0].at[pl.ds(0, TAIL)], sem_gd[0])
    g_s.wait()
    g_d.wait()

    @pl.loop(0, TAIL_GROUPS * LANES, step=LANES)
    def _tail_group(i0):
        _dot_group(rows_s[0], rows_d[0], out_v[0], i0)

    pltpu.sync_copy(out_v[0].at[pl.ds(0, TAIL)],
                    out_hbm.at[pl.ds(base_e + t_off, TAIL)])


def _make_sc_kernel():
    mesh = plsc.VectorSubcoreMesh(core_axis_name="c", subcore_axis_name="s")
    cp = pltpu.CompilerParams()
    if "needs_layout_passes" in pltpu.CompilerParams.__dataclass_fields__:
        cp = dataclasses.replace(cp, needs_layout_passes=False)
    scratch_types = [
        pltpu.VMEM_SHARED((N_NODES, DIMW), jnp.int32),
        pltpu.VMEM((B_W,), jnp.int32),
        pltpu.VMEM((B_W,), jnp.int32),
        pltpu.VMEM((CONV_CHUNK, DIM), jnp.float32),
        pltpu.VMEM((CONV_CHUNK, DIMW), jnp.int32),
    ]
    for _ in range(NBUF):
        scratch_types.append(pltpu.VMEM((CHUNK, DIMW), jnp.int32))
        scratch_types.append(pltpu.VMEM((CHUNK, DIMW), jnp.int32))
    scratch_types += [pltpu.VMEM((CHUNK,), jnp.float32)] * NBUF
    scratch_types += [pltpu.SemaphoreType.DMA] * (3 * NBUF)
    return pl.kernel(
        _sc_body,
        out_type=jax.ShapeDtypeStruct((N_EDGES,), jnp.float32),
        mesh=mesh,
        scratch_types=scratch_types,
        compiler_params=cp,
    )


_sc_kernel = _make_sc_kernel()


def kernel(z, edge_index):
    ei = edge_index.astype(jnp.int32)
    return _sc_kernel(z, ei[0], ei[1])


# speedup vs baseline: 1.4019x; 1.0021x over previous
import dataclasses

import jax
import jax.numpy as jnp
from jax import lax
from jax.experimental import pallas as pl
from jax.experimental.pallas import tpu as pltpu
from jax.experimental.pallas import tpu_sc as plsc

N_NODES = 10000
N_EDGES = 160000
DIM = 256
DIMW = DIM // 2
LANES = 16
N_WORKERS = 32
B_W = N_EDGES // N_WORKERS
CHUNK = 48
NBUF = 2
NF = B_W // CHUNK
TAIL = B_W - NF * CHUNK
TAIL_GROUPS = -(-TAIL // LANES)
CONV_CHUNK = 40
N_CONV = N_NODES // CONV_CHUNK


def _dot_group(rows_s, rows_d, out_ref, i0):
    out_ref[pl.ds(i0, LANES)] = jnp.zeros((LANES,), jnp.float32)
    base_idx = jnp.full((LANES,), i0, jnp.int32)
    for e in range(LANES):
        i = i0 + e
        acc0 = acc1 = None
        for cd in range(DIMW // LANES):
            s = plsc.bitcast(rows_s[i, pl.ds(cd * LANES, LANES)],
                             jnp.bfloat16)
            d = plsc.bitcast(rows_d[i, pl.ds(cd * LANES, LANES)],
                             jnp.bfloat16)
            pa, pb = plsc.unpack(s * d, format=plsc.PackFormat.INTERLEAVED,
                                 preferred_element_type=jnp.float32)
            acc0 = pa if acc0 is None else acc0 + pa
            acc1 = pb if acc1 is None else acc1 + pb
        plsc.addupdate_scatter(out_ref, [base_idx + e], acc0 + acc1)
    v = out_ref[pl.ds(i0, LANES)]
    out_ref[pl.ds(i0, LANES)] = 1.0 / (1.0 + jnp.exp(-v))


def _sc_body(z_hbm, srci_hbm, dsti_hbm, out_hbm,
             zbf_sh, idx_s, idx_d, stage_f, stage_o, *scr):
    rows_s = tuple(scr[2 * b] for b in range(NBUF))
    rows_d = tuple(scr[2 * b + 1] for b in range(NBUF))
    out_v = tuple(scr[2 * NBUF + b] for b in range(NBUF))
    sems = scr[3 * NBUF:]
    sem_gs = tuple(sems[2 * b] for b in range(NBUF))
    sem_gd = tuple(sems[2 * b + 1] for b in range(NBUF))
    sem_o = tuple(sems[2 * NBUF + b] for b in range(NBUF))

    cid = lax.axis_index("c")
    sid = lax.axis_index("s")
    wid = sid * 2 + cid
    base_e = wid * B_W

    n_conv = jnp.where(sid < N_CONV - (N_CONV // 16) * 16,
                       N_CONV // 16 + 1, N_CONV // 16)

    @pl.loop(0, n_conv)
    def _conv(j):
        r0 = pl.multiple_of((sid + j * 16) * CONV_CHUNK, 16)
        pltpu.sync_copy(z_hbm.at[pl.ds(r0, CONV_CHUNK)], stage_f)

        @pl.loop(0, CONV_CHUNK)
        def _conv_row(r):
            for cd in range(DIM // (2 * LANES)):
                a = stage_f[r, pl.ds(cd * 2 * LANES, LANES)]
                b = stage_f[r, pl.ds(cd * 2 * LANES + LANES, LANES)]
                p = plsc.pack(a, b, format=plsc.PackFormat.INTERLEAVED)
                stage_o[r, pl.ds(cd * LANES, LANES)] = plsc.bitcast(
                    p, jnp.int32)

        pltpu.sync_copy(stage_o, zbf_sh.at[pl.ds(r0, CONV_CHUNK)])
    plsc.subcore_barrier()

    pltpu.sync_copy(srci_hbm.at[pl.ds(base_e, B_W)], idx_s)
    pltpu.sync_copy(dsti_hbm.at[pl.ds(base_e, B_W)], idx_d)

    def start_gather(k, b):
        off = pl.multiple_of(k * CHUNK, 8)
        pltpu.async_copy(zbf_sh.at[idx_s.at[pl.ds(off, CHUNK)]],
                         rows_s[b], sem_gs[b])
        pltpu.async_copy(zbf_sh.at[idx_d.at[pl.ds(off, CHUNK)]],
                         rows_d[b], sem_gd[b])

    def wait_gather(b):
        pltpu.make_async_copy(zbf_sh.at[pl.ds(0, CHUNK)], rows_s[b],
                              sem_gs[b]).wait()
        pltpu.make_async_copy(zbf_sh.at[pl.ds(0, CHUNK)], rows_d[b],
                              sem_gd[b]).wait()

    def wait_store(b):
        pltpu.make_async_copy(out_v[b], out_hbm.at[pl.ds(0, CHUNK)],
                              sem_o[b]).wait()

    for b in range(NBUF):
        start_gather(b, b)

    @pl.loop(0, NF, step=NBUF)
    def _ring(k):
        for b in range(NBUF):
            kk = k + b
            wait_gather(b)
            @pl.when(kk >= NBUF)
            def _():
                wait_store(b)

            @pl.loop(0, CHUNK, step=LANES)
            def _group(i0):
                _dot_group(rows_s[b], rows_d[b], out_v[b], i0)

            off = pl.multiple_of(base_e + kk * CHUNK, 8)
            pltpu.async_copy(out_v[b], out_hbm.at[pl.ds(off, CHUNK)],
                             sem_o[b])

            @pl.when(kk + NBUF < NF)
            def _():
                start_gather(kk + NBUF, b)

    for b in range(NBUF):
        wait_store(b)

    t_off = NF * CHUNK
    g_s = pltpu.async_copy(zbf_sh.at[idx_s.at[pl.ds(t_off, TAIL)]],
                           rows_s[0].at[pl.ds(0, TAIL)], sem_gs[0])
    g_d = pltpu.async_copy(zbf_sh.at[idx_d.at[pl.ds(t_off, TAIL)]],
                           rows_d[0].at[pl.ds(0, TAIL)], sem_gd[0])
    g_s.wait()
    g_d.wait()

    @pl.loop(0, TAIL_GROUPS * LANES, step=LANES)
    def _tail_group(i0):
        _dot_group(rows_s[0], rows_d[0], out_v[0], i0)

    pltpu.sync_copy(out_v[0].at[pl.ds(0, TAIL)],
                    out_hbm.at[pl.ds(base_e + t_off, TAIL)])


def _make_sc_kernel():
    mesh = plsc.VectorSubcoreMesh(core_axis_name="c", subcore_axis_name="s")
    cp = pltpu.CompilerParams()
    if "needs_layout_passes" in pltpu.CompilerParams.__dataclass_fields__:
        cp = dataclasses.replace(cp, needs_layout_passes=False)
    scratch_types = [
        pltpu.VMEM_SHARED((N_NODES, DIMW), jnp.int32),
        pltpu.VMEM((B_W,), jnp.int32),
        pltpu.VMEM((B_W,), jnp.int32),
        pltpu.VMEM((CONV_CHUNK, DIM), jnp.float32),
        pltpu.VMEM((CONV_CHUNK, DIMW), jnp.int32),
    ]
    for _ in range(NBUF):
        scratch_types.append(pltpu.VMEM((CHUNK, DIMW), jnp.int32))
        scratch_types.append(pltpu.VMEM((CHUNK, DIMW), jnp.int32))
    scratch_types += [pltpu.VMEM((CHUNK,), jnp.float32)] * NBUF
    scratch_types += [pltpu.SemaphoreType.DMA] * (3 * NBUF)
    return pl.kernel(
        _sc_body,
        out_type=jax.ShapeDtypeStruct((N_EDGES,), jnp.float32),
        mesh=mesh,
        scratch_types=scratch_types,
        compiler_params=cp,
    )


_sc_kernel = _make_sc_kernel()


def kernel(z, edge_index):
    ei = edge_index.astype(jnp.int32)
    return _sc_kernel(z, ei[0], ei[1])
